# baseline (speedup 1.0000x reference)
import functools

import jax
import jax.numpy as jnp
from jax import lax
from jax.experimental import pallas as pl
from jax.experimental.pallas import tpu as pltpu
from jax.experimental.pallas import tpu_sc as plsc

N_NODES = 10000
N_EDGES = 160000
D = 128
DP = D // 2

N_PAD = 10112
E_PAD = 163840

NC, NS = 2, 16
NW = NC * NS
EPW = E_PAD // NW
CH = 64
NCH = EPW // CH
NSLOT = 4
CH_G = 40
NCH_G = EPW // CH_G
NSUP = NCH_G // NSLOT
NSLOT_S = 4
NSUP_S = NCH // NSLOT_S
RPT = N_PAD // NS

_SC_MESH = plsc.VectorSubcoreMesh(core_axis_name="c", subcore_axis_name="s")


def _pack_bf16(h):
    hb = h.astype(jnp.bfloat16)
    lo = lax.convert_element_type(
        lax.bitcast_convert_type(hb[:, :DP], jnp.uint16), jnp.uint32)
    hi = lax.convert_element_type(
        lax.bitcast_convert_type(hb[:, DP:], jnp.uint16), jnp.uint32)
    return lax.bitcast_convert_type(lo | (hi << 16), jnp.float32)


def _unpack_bf16(xp):
    w = lax.bitcast_convert_type(xp, jnp.uint32)
    lo = lax.bitcast_convert_type(
        lax.convert_element_type(w & jnp.uint32(0xFFFF), jnp.uint16),
        jnp.bfloat16)
    hi = lax.bitcast_convert_type(
        lax.convert_element_type(w >> jnp.uint32(16), jnp.uint16),
        jnp.bfloat16)
    return jnp.concatenate([lo, hi], axis=1)


_GATHER_SCRATCH = (
    [pltpu.VMEM((CH_G,), jnp.int32) for _ in range(2 * NSLOT)]
    + [pltpu.VMEM((CH_G, D), jnp.float32) for _ in range(2 * NSLOT)]
    + [pltpu.SemaphoreType.DMA for _ in range(2 * NSLOT)]
    + [pltpu.VMEM_SHARED((N_PAD, D), jnp.float32)]
)


@functools.partial(
    pl.kernel,
    out_type=(jax.ShapeDtypeStruct((E_PAD, D), jnp.float32),
              jax.ShapeDtypeStruct((E_PAD, D), jnp.float32)),
    mesh=_SC_MESH,
    scratch_types=_GATHER_SCRATCH,
)
def _sc_gather2(xh, src, dst, xs_out, xd_out, *scratch):
    si = scratch[0:NSLOT]
    di = scratch[NSLOT:2 * NSLOT]
    sr = scratch[2 * NSLOT:3 * NSLOT]
    dr = scratch[3 * NSLOT:4 * NSLOT]
    gsem = scratch[4 * NSLOT:5 * NSLOT]
    wsem = scratch[5 * NSLOT:6 * NSLOT]
    tbl = scratch[6 * NSLOT]

    sid = lax.axis_index("s")
    wid = sid * NC + lax.axis_index("c")
    base = wid * EPW

    pltpu.sync_copy(xh.at[pl.ds(sid * RPT, RPT)], tbl.at[pl.ds(sid * RPT, RPT)])
    plsc.subcore_barrier()

    def drain_writes():
        for j in range(NSLOT):
            pltpu.make_async_copy(sr[j], xs_out.at[pl.ds(0, CH_G)],
                                  wsem[j]).wait()
            pltpu.make_async_copy(dr[j], xd_out.at[pl.ds(0, CH_G)],
                                  wsem[j]).wait()

    def body(t, carry):
        @pl.when(t > 0)
        def _():
            drain_writes()
        cps = []
        for j in range(NSLOT):
            off = base + (t * NSLOT + j) * CH_G
            pltpu.sync_copy(src.at[pl.ds(off, CH_G)], si[j])
            pltpu.sync_copy(dst.at[pl.ds(off, CH_G)], di[j])
            cps.append(pltpu.async_copy(tbl.at[si[j]], sr[j], gsem[j]))
            cps.append(pltpu.async_copy(tbl.at[di[j]], dr[j], gsem[j]))
        for j in range(NSLOT):
            off = base + (t * NSLOT + j) * CH_G
            cps[2 * j].wait()
            cps[2 * j + 1].wait()
            pltpu.async_copy(sr[j], xs_out.at[pl.ds(off, CH_G)], wsem[j])
            pltpu.async_copy(dr[j], xd_out.at[pl.ds(off, CH_G)], wsem[j])
        return carry

    lax.fori_loop(0, NSUP, body, 0)
    drain_writes()


_SCATTER_SCRATCH = (
    [pltpu.VMEM((CH,), jnp.int32) for _ in range(NSLOT_S)]
    + [pltpu.VMEM((CH, D), jnp.float32) for _ in range(NSLOT_S)]
    + [pltpu.SemaphoreType.DMA for _ in range(NSLOT_S)]
    + [pltpu.VMEM_SHARED((N_PAD, D), jnp.float32)]
)


@functools.partial(
    pl.kernel,
    out_type=jax.ShapeDtypeStruct((NC, N_PAD, D), jnp.float32),
    mesh=_SC_MESH,
    scratch_types=_SCATTER_SCRATCH,
)
def _sc_scatter_add(eh, dst, zrows, agg_out, *scratch):
    di = scratch[0:NSLOT_S]
    rows = scratch[NSLOT_S:2 * NSLOT_S]
    lsem = scratch[2 * NSLOT_S:3 * NSLOT_S]
    acc = scratch[3 * NSLOT_S]

    cid = lax.axis_index("c")
    sid = lax.axis_index("s")
    wid = sid * NC + cid
    base = wid * EPW

    pltpu.sync_copy(zrows, acc.at[pl.ds(sid * RPT, RPT)])

    def fire(c, j):
        off = base + c * CH
        pltpu.async_copy(dst.at[pl.ds(off, CH)], di[j], lsem[j])
        pltpu.async_copy(eh.at[pl.ds(off, CH)], rows[j], lsem[j])

    def drain(j):
        pltpu.make_async_copy(dst.at[pl.ds(base, CH)], di[j], lsem[j]).wait()
        pltpu.make_async_copy(eh.at[pl.ds(base, CH)], rows[j], lsem[j]).wait()

    for j in range(NSLOT_S):
        fire(j, j)

    plsc.subcore_barrier()

    def body(t, carry):
        for j in range(NSLOT_S):
            c = t * NSLOT_S + j
            drain(j)
            pltpu.sync_copy(rows[j], acc.at[di[j]], add=True)

            @pl.when(c + NSLOT_S < NCH)
            def _():
                fire(c + NSLOT_S, j)
        return carry

    lax.fori_loop(0, NSUP_S, body, 0)
    plsc.subcore_barrier()

    pltpu.sync_copy(acc.at[pl.ds(sid * RPT, RPT)],
                    agg_out.at[cid, pl.ds(sid * RPT, RPT)])


def _mlp_body(n_in, packed, has_ln, residual, emit_packed):
    def body(*refs):
        xs = refs[:n_in]
        w1s = refs[n_in:2 * n_in]
        k = 2 * n_in
        b1, w2, b2, w3, b3 = refs[k:k + 5]
        k += 5
        if has_ln:
            g, bln = refs[k:k + 2]
            k += 2
        out = refs[k]
        h = None
        for j in range(n_in):
            xv = xs[j][...]
            if packed[j]:
                xv = _unpack_bf16(xv)
            d = jnp.dot(xv, w1s[j][...], preferred_element_type=jnp.float32)
            h = d if h is None else h + d
        h = jnp.maximum(h + b1[...], 0.0)
        h = jnp.maximum(
            jnp.dot(h, w2[...], preferred_element_type=jnp.float32) + b2[...],
            0.0)
        h = jnp.dot(h, w3[...], preferred_element_type=jnp.float32) + b3[...]
        if has_ln:
            mu = jnp.mean(h, axis=-1, keepdims=True)
            var = jnp.mean((h - mu) * (h - mu), axis=-1, keepdims=True)
            h = (h - mu) * lax.rsqrt(var + 1e-5) * g[...] + bln[...]
        if residual:
            h = xs[0][...] + h
        out[...] = h
        if emit_packed:
            refs[k + 1][...] = _pack_bf16(h)
    return body


def _mlp_call(inputs, w1s, b1, w2, b2, w3, b3, ln=None, residual=False,
              packed=None, emit_packed=False, rows=2048):
    m = inputs[0].shape[0]
    n_in = len(inputs)
    packed = packed or (False,) * n_in
    weights = list(w1s) + [b1, w2, b2, w3, b3]
    if ln is not None:
        weights += [ln[0], ln[1]]
    in_specs = [pl.BlockSpec((rows, a.shape[1]), lambda i: (i, 0))
                for a in inputs]
    in_specs += [pl.BlockSpec(w.shape, lambda i: (0, 0)) for w in weights]
    out_specs = pl.BlockSpec((rows, D), lambda i: (i, 0))
    out_shape = jax.ShapeDtypeStruct((m, D), jnp.float32)
    if emit_packed:
        out_specs = (out_specs, pl.BlockSpec((rows, DP), lambda i: (i, 0)))
        out_shape = (out_shape, jax.ShapeDtypeStruct((m, DP), jnp.float32))
    return pl.pallas_call(
        _mlp_body(n_in, packed, ln is not None, residual, emit_packed),
        grid=(m // rows,),
        in_specs=in_specs,
        out_specs=out_specs,
        out_shape=out_shape,
    )(*(list(inputs) + weights))


def _prep(p, pad_out=None):
    w3, b3 = p["l3"]["W"], p["l3"]["b"]
    if pad_out is not None:
        w3 = jnp.pad(w3, ((0, 0), (0, pad_out - w3.shape[1])))
        b3 = jnp.pad(b3, (0, pad_out - b3.shape[0]))
    out = dict(
        w1=p["l1"]["W"], b1=p["l1"]["b"][None, :],
        w2=p["l2"]["W"], b2=p["l2"]["b"][None, :],
        w3=w3, b3=b3[None, :],
    )
    if "ln_g" in p:
        out["ln"] = (p["ln_g"][None, :], p["ln_b"][None, :])
    return out


def kernel(x, edge_index, edge_attr, params):
    src = edge_index[0].astype(jnp.int32)
    dst = edge_index[1].astype(jnp.int32)

    x_p = jnp.pad(x, ((0, N_PAD - N_NODES), (0, 0)))
    ea_p = jnp.pad(edge_attr, ((0, E_PAD - N_EDGES), (0, 0)))
    pad_idx = N_NODES + (jnp.arange(E_PAD - N_EDGES, dtype=jnp.int32)
                         % (N_PAD - N_NODES))
    src_p = jnp.concatenate([src, pad_idx])
    dst_p = jnp.concatenate([dst, pad_idx])
    zrows = jnp.zeros((RPT, D), jnp.float32)

    ne = _prep(params["node_enc"])
    xh = _mlp_call([x_p], [ne["w1"]], ne["b1"], ne["w2"], ne["b2"],
                   ne["w3"], ne["b3"], ln=ne["ln"], rows=1264)
    ee = _prep(params["edge_enc"])
    eh = _mlp_call([ea_p], [ee["w1"]], ee["b1"], ee["w2"], ee["b2"],
                   ee["w3"], ee["b3"], ln=ee["ln"])

    for blk in params["blocks"]:
        eb = _prep(blk["eb"])
        nb = _prep(blk["nb"])
        xs_g, xd_g = _sc_gather2(xh, src_p, dst_p)
        eh = _mlp_call(
            [eh, xs_g, xd_g],
            [eb["w1"][0:D], eb["w1"][D:2 * D], eb["w1"][2 * D:3 * D]],
            eb["b1"], eb["w2"], eb["b2"], eb["w3"], eb["b3"],
            ln=eb["ln"], residual=True)
        agg2 = _sc_scatter_add(eh, dst_p, zrows)
        xh = _mlp_call(
            [xh, agg2[0], agg2[1]],
            [nb["w1"][0:D], nb["w1"][D:2 * D], nb["w1"][D:2 * D]],
            nb["b1"], nb["w2"], nb["b2"], nb["w3"], nb["b3"],
            ln=nb["ln"], residual=True, rows=1264)

    de = _prep(params["dec"], pad_out=D)
    out = _mlp_call([xh], [de["w1"]], de["b1"], de["w2"], de["b2"],
                    de["w3"], de["b3"], rows=1264)
    return out[:N_NODES, :3]


# speedup vs baseline: 1.1011x; 1.1011x over previous
import functools

import jax
import jax.numpy as jnp
from jax import lax
from jax.experimental import pallas as pl
from jax.experimental.pallas import tpu as pltpu
from jax.experimental.pallas import tpu_sc as plsc

N_NODES = 10000
N_EDGES = 160000
D = 128
DP = D // 2

N_PAD = 10112
E_PAD = 163840

NC, NS = 2, 16
NW = NC * NS
EPW = E_PAD // NW
CH = 64
NCH = EPW // CH
NSLOT = 2
CH_G = 80
NCH_G = EPW // CH_G
NSUP = NCH_G // NSLOT
NSLOT_S = 4
NSUP_S = NCH // NSLOT_S
RPT = N_PAD // NS

_SC_MESH = plsc.VectorSubcoreMesh(core_axis_name="c", subcore_axis_name="s")


def _pack_bf16(h):
    hb = h.astype(jnp.bfloat16)
    lo = lax.convert_element_type(
        lax.bitcast_convert_type(hb[:, :DP], jnp.uint16), jnp.uint32)
    hi = lax.convert_element_type(
        lax.bitcast_convert_type(hb[:, DP:], jnp.uint16), jnp.uint32)
    return lax.bitcast_convert_type(lo | (hi << 16), jnp.float32)


def _unpack_bf16(xp):
    w = lax.bitcast_convert_type(xp, jnp.uint32)
    lo = lax.bitcast_convert_type(
        lax.convert_element_type(w & jnp.uint32(0xFFFF), jnp.uint16),
        jnp.bfloat16)
    hi = lax.bitcast_convert_type(
        lax.convert_element_type(w >> jnp.uint32(16), jnp.uint16),
        jnp.bfloat16)
    return jnp.concatenate([lo, hi], axis=1)


_GATHER_SCRATCH = (
    [pltpu.VMEM((CH_G,), jnp.int32) for _ in range(2 * NSLOT)]
    + [pltpu.VMEM((CH_G, D), jnp.float32) for _ in range(2 * NSLOT)]
    + [pltpu.SemaphoreType.DMA for _ in range(2 * NSLOT)]
    + [pltpu.VMEM_SHARED((N_PAD, D), jnp.float32)]
)


@functools.partial(
    pl.kernel,
    out_type=(jax.ShapeDtypeStruct((E_PAD, D), jnp.float32),
              jax.ShapeDtypeStruct((E_PAD, D), jnp.float32)),
    mesh=_SC_MESH,
    scratch_types=_GATHER_SCRATCH,
)
def _sc_gather2(xh, src, dst, xs_out, xd_out, *scratch):
    si = scratch[0:NSLOT]
    di = scratch[NSLOT:2 * NSLOT]
    sr = scratch[2 * NSLOT:3 * NSLOT]
    dr = scratch[3 * NSLOT:4 * NSLOT]
    gsem = scratch[4 * NSLOT:5 * NSLOT]
    wsem = scratch[5 * NSLOT:6 * NSLOT]
    tbl = scratch[6 * NSLOT]

    sid = lax.axis_index("s")
    wid = sid * NC + lax.axis_index("c")
    base = wid * EPW

    pltpu.sync_copy(xh.at[pl.ds(sid * RPT, RPT)], tbl.at[pl.ds(sid * RPT, RPT)])
    plsc.subcore_barrier()

    def drain_writes():
        for j in range(NSLOT):
            pltpu.make_async_copy(sr[j], xs_out.at[pl.ds(0, CH_G)],
                                  wsem[j]).wait()
            pltpu.make_async_copy(dr[j], xd_out.at[pl.ds(0, CH_G)],
                                  wsem[j]).wait()

    def body(t, carry):
        @pl.when(t > 0)
        def _():
            drain_writes()
        cps = []
        for j in range(NSLOT):
            off = base + (t * NSLOT + j) * CH_G
            pltpu.sync_copy(src.at[pl.ds(off, CH_G)], si[j])
            pltpu.sync_copy(dst.at[pl.ds(off, CH_G)], di[j])
            cps.append(pltpu.async_copy(tbl.at[si[j]], sr[j], gsem[j]))
            cps.append(pltpu.async_copy(tbl.at[di[j]], dr[j], gsem[j]))
        for j in range(NSLOT):
            off = base + (t * NSLOT + j) * CH_G
            cps[2 * j].wait()
            cps[2 * j + 1].wait()
            pltpu.async_copy(sr[j], xs_out.at[pl.ds(off, CH_G)], wsem[j])
            pltpu.async_copy(dr[j], xd_out.at[pl.ds(off, CH_G)], wsem[j])
        return carry

    lax.fori_loop(0, NSUP, body, 0)
    drain_writes()


_SCATTER_SCRATCH = (
    [pltpu.VMEM((CH,), jnp.int32) for _ in range(NSLOT_S)]
    + [pltpu.VMEM((CH, D), jnp.float32) for _ in range(NSLOT_S)]
    + [pltpu.SemaphoreType.DMA for _ in range(NSLOT_S)]
    + [pltpu.VMEM_SHARED((N_PAD, D), jnp.float32)]
)


@functools.partial(
    pl.kernel,
    out_type=jax.ShapeDtypeStruct((NC, N_PAD, D), jnp.float32),
    mesh=_SC_MESH,
    scratch_types=_SCATTER_SCRATCH,
)
def _sc_scatter_add(eh, dst, zrows, agg_out, *scratch):
    di = scratch[0:NSLOT_S]
    rows = scratch[NSLOT_S:2 * NSLOT_S]
    lsem = scratch[2 * NSLOT_S:3 * NSLOT_S]
    acc = scratch[3 * NSLOT_S]

    cid = lax.axis_index("c")
    sid = lax.axis_index("s")
    wid = sid * NC + cid
    base = wid * EPW

    pltpu.sync_copy(zrows, acc.at[pl.ds(sid * RPT, RPT)])

    def fire(c, j):
        off = base + c * CH
        pltpu.async_copy(dst.at[pl.ds(off, CH)], di[j], lsem[j])
        pltpu.async_copy(eh.at[pl.ds(off, CH)], rows[j], lsem[j])

    def drain(j):
        pltpu.make_async_copy(dst.at[pl.ds(base, CH)], di[j], lsem[j]).wait()
        pltpu.make_async_copy(eh.at[pl.ds(base, CH)], rows[j], lsem[j]).wait()

    for j in range(NSLOT_S):
        fire(j, j)

    plsc.subcore_barrier()

    def body(t, carry):
        for j in range(NSLOT_S):
            c = t * NSLOT_S + j
            drain(j)
            pltpu.sync_copy(rows[j], acc.at[di[j]], add=True)

            @pl.when(c + NSLOT_S < NCH)
            def _():
                fire(c + NSLOT_S, j)
        return carry

    lax.fori_loop(0, NSUP_S, body, 0)
    plsc.subcore_barrier()

    pltpu.sync_copy(acc.at[pl.ds(sid * RPT, RPT)],
                    agg_out.at[cid, pl.ds(sid * RPT, RPT)])


def _mlp_body(n_in, packed, has_ln, residual, emit_packed):
    def body(*refs):
        xs = refs[:n_in]
        w1s = refs[n_in:2 * n_in]
        k = 2 * n_in
        b1, w2, b2, w3, b3 = refs[k:k + 5]
        k += 5
        if has_ln:
            g, bln = refs[k:k + 2]
            k += 2
        out = refs[k]
        h = None
        for j in range(n_in):
            xv = xs[j][...]
            if packed[j]:
                xv = _unpack_bf16(xv)
            d = jnp.dot(xv, w1s[j][...], preferred_element_type=jnp.float32)
            h = d if h is None else h + d
        h = jnp.maximum(h + b1[...], 0.0)
        h = jnp.maximum(
            jnp.dot(h, w2[...], preferred_element_type=jnp.float32) + b2[...],
            0.0)
        h = jnp.dot(h, w3[...], preferred_element_type=jnp.float32) + b3[...]
        if has_ln:
            mu = jnp.mean(h, axis=-1, keepdims=True)
            var = jnp.mean((h - mu) * (h - mu), axis=-1, keepdims=True)
            h = (h - mu) * lax.rsqrt(var + 1e-5) * g[...] + bln[...]
        if residual:
            h = xs[0][...] + h
        out[...] = h
        if emit_packed:
            refs[k + 1][...] = _pack_bf16(h)
    return body


def _mlp_call(inputs, w1s, b1, w2, b2, w3, b3, ln=None, residual=False,
              packed=None, emit_packed=False, rows=2048):
    m = inputs[0].shape[0]
    n_in = len(inputs)
    packed = packed or (False,) * n_in
    weights = list(w1s) + [b1, w2, b2, w3, b3]
    if ln is not None:
        weights += [ln[0], ln[1]]
    in_specs = [pl.BlockSpec((rows, a.shape[1]), lambda i: (i, 0))
                for a in inputs]
    in_specs += [pl.BlockSpec(w.shape, lambda i: (0, 0)) for w in weights]
    out_specs = pl.BlockSpec((rows, D), lambda i: (i, 0))
    out_shape = jax.ShapeDtypeStruct((m, D), jnp.float32)
    if emit_packed:
        out_specs = (out_specs, pl.BlockSpec((rows, DP), lambda i: (i, 0)))
        out_shape = (out_shape, jax.ShapeDtypeStruct((m, DP), jnp.float32))
    return pl.pallas_call(
        _mlp_body(n_in, packed, ln is not None, residual, emit_packed),
        grid=(m // rows,),
        in_specs=in_specs,
        out_specs=out_specs,
        out_shape=out_shape,
    )(*(list(inputs) + weights))


def _prep(p, pad_out=None):
    w3, b3 = p["l3"]["W"], p["l3"]["b"]
    if pad_out is not None:
        w3 = jnp.pad(w3, ((0, 0), (0, pad_out - w3.shape[1])))
        b3 = jnp.pad(b3, (0, pad_out - b3.shape[0]))
    out = dict(
        w1=p["l1"]["W"], b1=p["l1"]["b"][None, :],
        w2=p["l2"]["W"], b2=p["l2"]["b"][None, :],
        w3=w3, b3=b3[None, :],
    )
    if "ln_g" in p:
        out["ln"] = (p["ln_g"][None, :], p["ln_b"][None, :])
    return out


def kernel(x, edge_index, edge_attr, params):
    src = edge_index[0].astype(jnp.int32)
    dst = edge_index[1].astype(jnp.int32)

    x_p = jnp.pad(x, ((0, N_PAD - N_NODES), (0, 0)))
    ea_p = jnp.pad(edge_attr, ((0, E_PAD - N_EDGES), (0, 0)))
    pad_idx = N_NODES + (jnp.arange(E_PAD - N_EDGES, dtype=jnp.int32)
                         % (N_PAD - N_NODES))
    src_p = jnp.concatenate([src, pad_idx])
    dst_p = jnp.concatenate([dst, pad_idx])
    zrows = jnp.zeros((RPT, D), jnp.float32)

    ne = _prep(params["node_enc"])
    xh = _mlp_call([x_p], [ne["w1"]], ne["b1"], ne["w2"], ne["b2"],
                   ne["w3"], ne["b3"], ln=ne["ln"], rows=1264)
    ee = _prep(params["edge_enc"])
    eh = _mlp_call([ea_p], [ee["w1"]], ee["b1"], ee["w2"], ee["b2"],
                   ee["w3"], ee["b3"], ln=ee["ln"])

    for blk in params["blocks"]:
        eb = _prep(blk["eb"])
        nb = _prep(blk["nb"])
        xs_g, xd_g = _sc_gather2(xh, src_p, dst_p)
        eh = _mlp_call(
            [eh, xs_g, xd_g],
            [eb["w1"][0:D], eb["w1"][D:2 * D], eb["w1"][2 * D:3 * D]],
            eb["b1"], eb["w2"], eb["b2"], eb["w3"], eb["b3"],
            ln=eb["ln"], residual=True)
        agg2 = _sc_scatter_add(eh, dst_p, zrows)
        xh = _mlp_call(
            [xh, agg2[0], agg2[1]],
            [nb["w1"][0:D], nb["w1"][D:2 * D], nb["w1"][D:2 * D]],
            nb["b1"], nb["w2"], nb["b2"], nb["w3"], nb["b3"],
            ln=nb["ln"], residual=True, rows=1264)

    de = _prep(params["dec"], pad_out=D)
    out = _mlp_call([xh], [de["w1"]], de["b1"], de["w2"], de["b2"],
                    de["w3"], de["b3"], rows=1264)
    return out[:N_NODES, :3]
